# Initial kernel scaffold; baseline (speedup 1.0000x reference)
import jax
import jax.numpy as jnp
from jax.experimental import pallas as pl


def kernel(inputs, tables):
    raise NotImplementedError("write your pallas kernel here")


# speedup vs baseline: 1.0139x; 1.0139x over previous
import functools

import jax
import jax.numpy as jnp
from jax import lax
from jax.experimental import pallas as pl
from jax.experimental.pallas import tpu as pltpu
from jax.experimental.pallas import tpu_sc as plsc

N_FIELDS = 26
VOCAB = 100000
EMBED_DIM = 32
BATCH = 4096

NC, NS, L = 2, 16, 16
NW = NC * NS
B_TOTAL = BATCH * N_FIELDS
B_PER_W = B_TOTAL // NW
ROWS_PER_GATHER = 128
G_PER_W = B_PER_W // ROWS_PER_GATHER
SL_PER_ROW = ROWS_PER_GATHER // L


def _sc_body(idx_hbm, tab_hbm, out_hbm, idx_v, rows_v, sem):
    wid = lax.axis_index("s") * NC + lax.axis_index("c")
    base = wid * B_PER_W
    pltpu.sync_copy(idx_hbm.at[pl.ds(base, B_PER_W)], idx_v)

    lane = lax.iota(jnp.int32, L)
    for k in range(B_PER_W // L):
        field = lax.rem(lane + (k * L) % N_FIELDS, N_FIELDS)
        sl = pl.ds(k * L, L)
        idx_v[sl] = idx_v[sl] + field * VOCAB

    copies = [
        pltpu.async_copy(
            tab_hbm.at[idx_v.at[pl.ds(j * ROWS_PER_GATHER, ROWS_PER_GATHER)]],
            rows_v.at[pl.ds(j * ROWS_PER_GATHER, ROWS_PER_GATHER)],
            sem,
        )
        for j in range(G_PER_W)
    ]
    for cp in copies:
        cp.wait()

    pltpu.sync_copy(rows_v, out_hbm.at[pl.ds(wid * B_PER_W, B_PER_W)])


@jax.jit
def _field_embed(idx2d, tab2d):
    run = functools.partial(
        pl.kernel,
        out_type=jax.ShapeDtypeStruct((B_TOTAL, EMBED_DIM), jnp.float32),
        mesh=plsc.VectorSubcoreMesh(core_axis_name="c", subcore_axis_name="s"),
        scratch_types=[
            pltpu.VMEM((B_PER_W,), jnp.int32),
            pltpu.VMEM((B_PER_W, EMBED_DIM), jnp.float32),
            pltpu.SemaphoreType.DMA,
        ],
        compiler_params=pltpu.CompilerParams(use_tc_tiling_on_sc=False),
    )
    return run(_sc_body)(idx2d, tab2d)


def kernel(inputs, tables):
    idx1d = inputs.astype(jnp.int32).reshape(B_TOTAL)
    tab2d = tables.reshape(N_FIELDS * VOCAB, EMBED_DIM)
    out = _field_embed(idx1d, tab2d)
    return out.reshape(BATCH, N_FIELDS, EMBED_DIM)
